# baseline (speedup 1.0000x reference)
import jax
import jax.numpy as jnp
from jax.experimental import pallas as pl
from jax.experimental.pallas import tpu as pltpu

_BATCH = 128
_DIM = 64
_CTX = 16
_MEM = 500000
_CHUNK = 2000


def _attn_body(q_ref, c_ref, k_ref, v_ref, mc_ref, ts_ref, o_ref,
               m_ref, l_ref, acc_ref):
    i = pl.program_id(0)

    @pl.when(i == 0)
    def _init():
        m_ref[...] = jnp.full_like(m_ref, -jnp.inf)
        l_ref[...] = jnp.zeros_like(l_ref)
        acc_ref[...] = jnp.zeros_like(acc_ref)

    s = jax.lax.dot_general(
        q_ref[...].astype(jnp.bfloat16), k_ref[...].astype(jnp.bfloat16),
        (((1,), (1,)), ((), ())), preferred_element_type=jnp.float32)
    s = s + 0.5 * jax.lax.dot_general(
        c_ref[...].astype(jnp.bfloat16), mc_ref[...].astype(jnp.bfloat16),
        (((1,), (1,)), ((), ())), preferred_element_type=jnp.float32)
    s = s + 0.3 * jnp.exp(0.1 * ts_ref[0])

    m_prev = m_ref[...]
    m_new = jnp.maximum(m_prev, jnp.max(s, axis=1, keepdims=True))
    alpha = jnp.exp(m_prev - m_new)
    p = jnp.exp(s - m_new)
    m_ref[...] = m_new
    l_ref[...] = l_ref[...] * alpha + jnp.sum(p, axis=1, keepdims=True)
    acc_ref[...] = acc_ref[...] * alpha + jax.lax.dot_general(
        p.astype(jnp.bfloat16), v_ref[...].astype(jnp.bfloat16),
        (((1,), (0,)), ((), ())), preferred_element_type=jnp.float32)

    @pl.when(i == pl.num_programs(0) - 1)
    def _fin():
        o_ref[...] = acc_ref[...] / l_ref[...]


def kernel(query, context, mem_keys, mem_values, mem_contexts, mem_timestamps):
    ts_row = mem_timestamps.reshape(_MEM // _CHUNK, 1, _CHUNK)
    return pl.pallas_call(
        _attn_body,
        grid=(_MEM // _CHUNK,),
        in_specs=[
            pl.BlockSpec((_BATCH, _DIM), lambda i: (0, 0)),
            pl.BlockSpec((_BATCH, _CTX), lambda i: (0, 0)),
            pl.BlockSpec((_CHUNK, _DIM), lambda i: (i, 0)),
            pl.BlockSpec((_CHUNK, _DIM), lambda i: (i, 0)),
            pl.BlockSpec((_CHUNK, _CTX), lambda i: (i, 0)),
            pl.BlockSpec((1, 1, _CHUNK), lambda i: (i, 0, 0)),
        ],
        out_specs=pl.BlockSpec((_BATCH, _DIM), lambda i: (0, 0)),
        out_shape=jax.ShapeDtypeStruct((_BATCH, _DIM), jnp.float32),
        scratch_shapes=[
            pltpu.VMEM((_BATCH, 1), jnp.float32),
            pltpu.VMEM((_BATCH, 1), jnp.float32),
            pltpu.VMEM((_BATCH, _DIM), jnp.float32),
        ],
    )(query, context, mem_keys, mem_values, mem_contexts, ts_row)


# speedup vs baseline: 1.1787x; 1.1787x over previous
import jax
import jax.numpy as jnp
from jax.experimental import pallas as pl
from jax.experimental.pallas import tpu as pltpu

_BATCH = 128
_DIM = 64
_CTX = 16
_MEM = 500000
_CHUNK = 10000


def _attn_body(q_ref, c_ref, k_ref, v_ref, mc_ref, ts_ref, o_ref,
               m_ref, l_ref, acc_ref):
    i = pl.program_id(0)

    @pl.when(i == 0)
    def _init():
        m_ref[...] = jnp.full_like(m_ref, -jnp.inf)
        l_ref[...] = jnp.zeros_like(l_ref)
        acc_ref[...] = jnp.zeros_like(acc_ref)

    s = jax.lax.dot_general(
        q_ref[...].astype(jnp.bfloat16), k_ref[...].astype(jnp.bfloat16),
        (((1,), (1,)), ((), ())), preferred_element_type=jnp.float32)
    s = s + 0.5 * jax.lax.dot_general(
        c_ref[...].astype(jnp.bfloat16), mc_ref[...].astype(jnp.bfloat16),
        (((1,), (1,)), ((), ())), preferred_element_type=jnp.float32)
    s = s + 0.3 * jnp.exp(0.1 * ts_ref[0])

    m_prev = m_ref[...]
    m_new = jnp.maximum(m_prev, jnp.max(s, axis=1, keepdims=True))
    alpha = jnp.exp(m_prev - m_new)
    p = jnp.exp(s - m_new)
    m_ref[...] = m_new
    l_ref[...] = l_ref[...] * alpha + jnp.sum(p, axis=1, keepdims=True)
    acc_ref[...] = acc_ref[...] * alpha + jax.lax.dot_general(
        p.astype(jnp.bfloat16), v_ref[...].astype(jnp.bfloat16),
        (((1,), (0,)), ((), ())), preferred_element_type=jnp.float32)

    @pl.when(i == pl.num_programs(0) - 1)
    def _fin():
        o_ref[...] = acc_ref[...] / l_ref[...]


def kernel(query, context, mem_keys, mem_values, mem_contexts, mem_timestamps):
    ts_row = mem_timestamps.reshape(_MEM // _CHUNK, 1, _CHUNK)
    return pl.pallas_call(
        _attn_body,
        grid=(_MEM // _CHUNK,),
        in_specs=[
            pl.BlockSpec((_BATCH, _DIM), lambda i: (0, 0)),
            pl.BlockSpec((_BATCH, _CTX), lambda i: (0, 0)),
            pl.BlockSpec((_CHUNK, _DIM), lambda i: (i, 0)),
            pl.BlockSpec((_CHUNK, _DIM), lambda i: (i, 0)),
            pl.BlockSpec((_CHUNK, _CTX), lambda i: (i, 0)),
            pl.BlockSpec((1, 1, _CHUNK), lambda i: (i, 0, 0)),
        ],
        out_specs=pl.BlockSpec((_BATCH, _DIM), lambda i: (0, 0)),
        out_shape=jax.ShapeDtypeStruct((_BATCH, _DIM), jnp.float32),
        scratch_shapes=[
            pltpu.VMEM((_BATCH, 1), jnp.float32),
            pltpu.VMEM((_BATCH, 1), jnp.float32),
            pltpu.VMEM((_BATCH, _DIM), jnp.float32),
        ],
    )(query, context, mem_keys, mem_values, mem_contexts, ts_row)
